# Initial kernel scaffold; baseline (speedup 1.0000x reference)
import jax
import jax.numpy as jnp
from jax.experimental import pallas as pl


def kernel(ids, cos_0, sin_0, cos_1, sin_1, cos_2, sin_2):
    raise NotImplementedError("write your pallas kernel here")


# speedup vs baseline: 13.6846x; 13.6846x over previous
import functools

import jax
import jax.numpy as jnp
from jax import lax
from jax.experimental import pallas as pl
from jax.experimental.pallas import tpu as pltpu
from jax.experimental.pallas import tpu_sc as plsc

_N_TOKENS = 32768
_WIDTHS = (16, 24, 24)
_COS_OFF = (0, 16, 40)
_SIN_OFF = (64, 80, 104)
_OUT_D = 128

_NUM_WORKERS = 32
_TOK_PER_W = _N_TOKENS // _NUM_WORKERS
_CHUNK = 128
_CHUNKS_PER_W = _TOK_PER_W // _CHUNK


def _body(ids_hbm, cos_0, sin_0, cos_1, sin_1, cos_2, sin_2, out_hbm,
          idx_v, b_c0, b_c1, b_c2, b_s0, b_s1, b_s2, gsem):
    cos_tabs = (cos_0, cos_1, cos_2)
    sin_tabs = (sin_0, sin_1, sin_2)
    cos_bufs = (b_c0, b_c1, b_c2)
    sin_bufs = (b_s0, b_s1, b_s2)

    c = lax.axis_index("c")
    s = lax.axis_index("s")
    wid = s * 2 + c
    base = wid * _TOK_PER_W

    pltpu.sync_copy(ids_hbm.at[:, pl.ds(wid * _CHUNKS_PER_W, _CHUNKS_PER_W), :],
                    idx_v)

    def chunk_body(j, carry):
        row0 = base + j * _CHUNK
        for a in range(3):
            pltpu.async_copy(cos_tabs[a].at[idx_v.at[a, j]], cos_bufs[a], gsem)
            pltpu.async_copy(sin_tabs[a].at[idx_v.at[a, j]], sin_bufs[a], gsem)
        for a in range(3):
            pltpu.make_async_copy(cos_tabs[a].at[idx_v.at[a, j]],
                                  cos_bufs[a], gsem).wait()
            pltpu.make_async_copy(sin_tabs[a].at[idx_v.at[a, j]],
                                  sin_bufs[a], gsem).wait()
        for a in range(3):
            pltpu.sync_copy(
                cos_bufs[a],
                out_hbm.at[pl.ds(row0, _CHUNK), pl.ds(_COS_OFF[a], _WIDTHS[a])])
            pltpu.sync_copy(
                sin_bufs[a],
                out_hbm.at[pl.ds(row0, _CHUNK), pl.ds(_SIN_OFF[a], _WIDTHS[a])])
        return carry

    lax.fori_loop(0, _CHUNKS_PER_W, chunk_body, 0)


@jax.jit
def kernel(ids, cos_0, sin_0, cos_1, sin_1, cos_2, sin_2):
    ids_r = jnp.transpose(ids.astype(jnp.int32)).reshape(
        3, _N_TOKENS // _CHUNK, _CHUNK)

    mesh = plsc.VectorSubcoreMesh(core_axis_name="c", subcore_axis_name="s")
    run = pl.kernel(
        _body,
        out_type=jax.ShapeDtypeStruct((_N_TOKENS, _OUT_D), jnp.float32),
        mesh=mesh,
        scratch_types=[
            pltpu.VMEM((3, _CHUNKS_PER_W, _CHUNK), jnp.int32),
            pltpu.VMEM((_CHUNK, 16), jnp.float32),
            pltpu.VMEM((_CHUNK, 24), jnp.float32),
            pltpu.VMEM((_CHUNK, 24), jnp.float32),
            pltpu.VMEM((_CHUNK, 16), jnp.float32),
            pltpu.VMEM((_CHUNK, 24), jnp.float32),
            pltpu.VMEM((_CHUNK, 24), jnp.float32),
            pltpu.SemaphoreType.DMA,
        ],
        compiler_params=pltpu.CompilerParams(use_tc_tiling_on_sc=False),
    )
    return run(ids_r, cos_0, sin_0, cos_1, sin_1, cos_2, sin_2)


# speedup vs baseline: 15.0622x; 1.1007x over previous
import functools

import jax
import jax.numpy as jnp
from jax import lax
from jax.experimental import pallas as pl
from jax.experimental.pallas import tpu as pltpu
from jax.experimental.pallas import tpu_sc as plsc

_N_TOKENS = 32768
_WIDTHS = (16, 24, 24)
_COS_OFF = (0, 16, 40)
_SIN_OFF = (64, 80, 104)
_OUT_D = 128

_NUM_WORKERS = 32
_TOK_PER_W = _N_TOKENS // _NUM_WORKERS
_CHUNK = 128
_CHUNKS_PER_W = _TOK_PER_W // _CHUNK


def _body(ids_hbm, cos_0, sin_0, cos_1, sin_1, cos_2, sin_2, out_hbm,
          idx_v, bufs0, bufs1, gsem, ssem):
    tabs = (cos_0, sin_0, cos_1, sin_1, cos_2, sin_2)
    offs = (_COS_OFF[0], _SIN_OFF[0], _COS_OFF[1], _SIN_OFF[1],
            _COS_OFF[2], _SIN_OFF[2])
    axes = (0, 0, 1, 1, 2, 2)
    wids = (16, 16, 24, 24, 24, 24)
    buf_sets = (bufs0, bufs1)

    c = lax.axis_index("c")
    s = lax.axis_index("s")
    wid = s * 2 + c
    base = wid * _TOK_PER_W

    pltpu.sync_copy(ids_hbm.at[:, pl.ds(wid * _CHUNKS_PER_W, _CHUNKS_PER_W), :],
                    idx_v)

    def gathers(j, bset):
        for t in range(6):
            pltpu.async_copy(tabs[t].at[idx_v.at[axes[t], j]], bset[t], gsem)

    def wait_gathers(j, bset):
        for t in range(6):
            pltpu.make_async_copy(tabs[t].at[idx_v.at[axes[t], j]],
                                  bset[t], gsem).wait()

    def out_slice(j, t):
        return out_hbm.at[pl.ds(base + j * _CHUNK, _CHUNK),
                          pl.ds(offs[t], wids[t])]

    def scatters(j, bset):
        for t in range(6):
            pltpu.async_copy(bset[t], out_slice(j, t), ssem)

    def wait_scatters(j, bset):
        for t in range(6):
            pltpu.make_async_copy(bset[t], out_slice(j, t), ssem).wait()

    gathers(0, buf_sets[0])
    for j in range(_CHUNKS_PER_W):
        bset = buf_sets[j % 2]
        if j + 1 < _CHUNKS_PER_W:
            if j >= 1:
                wait_scatters(j - 1, buf_sets[(j + 1) % 2])
            gathers(j + 1, buf_sets[(j + 1) % 2])
        wait_gathers(j, bset)
        scatters(j, bset)
    wait_scatters(_CHUNKS_PER_W - 2, buf_sets[(_CHUNKS_PER_W - 2) % 2])
    wait_scatters(_CHUNKS_PER_W - 1, buf_sets[(_CHUNKS_PER_W - 1) % 2])


@jax.jit
def kernel(ids, cos_0, sin_0, cos_1, sin_1, cos_2, sin_2):
    ids_r = jnp.transpose(ids.astype(jnp.int32)).reshape(
        3, _N_TOKENS // _CHUNK, _CHUNK)

    mesh = plsc.VectorSubcoreMesh(core_axis_name="c", subcore_axis_name="s")
    run = pl.kernel(
        _body,
        out_type=jax.ShapeDtypeStruct((_N_TOKENS, _OUT_D), jnp.float32),
        mesh=mesh,
        scratch_types=[
            pltpu.VMEM((3, _CHUNKS_PER_W, _CHUNK), jnp.int32),
            tuple(pltpu.VMEM((_CHUNK, w), jnp.float32)
                  for w in (16, 16, 24, 24, 24, 24)),
            tuple(pltpu.VMEM((_CHUNK, w), jnp.float32)
                  for w in (16, 16, 24, 24, 24, 24)),
            pltpu.SemaphoreType.DMA,
            pltpu.SemaphoreType.DMA,
        ],
        compiler_params=pltpu.CompilerParams(use_tc_tiling_on_sc=False),
    )
    return run(ids_r, cos_0, sin_0, cos_1, sin_1, cos_2, sin_2)
